# Initial kernel scaffold; baseline (speedup 1.0000x reference)
import jax
import jax.numpy as jnp
from jax.experimental import pallas as pl


def kernel(inputs, embedding):
    raise NotImplementedError("write your pallas kernel here")


# speedup vs baseline: 2.3135x; 2.3135x over previous
import functools

import jax
import jax.numpy as jnp
from jax.experimental import pallas as pl
from jax.experimental.pallas import tpu as pltpu

_K = 1024
_D = 64
_B = 16
_HC = 4
_T = 1024 // _HC
_N = 16384
_CC = 0.25


def _vq_kernel(x_ref, emb_ref, embT_ref,
               loss_ref, q_ref, perp_ref, enc_ref,
               sse_ref, cnt_ref):
    b = pl.program_id(0)
    hc = pl.program_id(1)

    @pl.when(jnp.logical_and(b == 0, hc == 0))
    def _init():
        sse_ref[...] = jnp.zeros_like(sse_ref)
        cnt_ref[...] = jnp.zeros_like(cnt_ref)

    x = x_ref[0].reshape(_D, _T)
    emb = emb_ref[...]

    x2 = jnp.sum(x * x, axis=0, keepdims=True)
    e2 = jnp.sum(emb * emb, axis=1, keepdims=True)
    mm = jax.lax.dot_general(emb, x, (((1,), (0,)), ((), ())),
                             preferred_element_type=jnp.float32)
    dist = (x2 + e2) - 2.0 * mm

    minv = jnp.min(dist, axis=0, keepdims=True)
    iota_k = jax.lax.broadcasted_iota(jnp.int32, (_K, _T), 0)
    idx = jnp.min(jnp.where(dist == minv, iota_k, _K),
                  axis=0, keepdims=True)

    onehot_T = (iota_k == idx).astype(jnp.float32)

    qT = jax.lax.dot_general(embT_ref[...], onehot_T, (((1,), (0,)), ((), ())),
                             preferred_element_type=jnp.float32)
    q_ref[0] = qT.reshape(_D, 32 // _HC, 32)

    idx_col = jnp.transpose(idx)
    iota_t = jax.lax.broadcasted_iota(jnp.int32, (_T, _K), 1)
    enc_ref[...] = (iota_t == idx_col).astype(jnp.float32)

    sse_ref[...] += minv
    cnt_ref[...] += jnp.sum(onehot_T, axis=1, keepdims=True)

    @pl.when(jnp.logical_and(b == _B - 1, hc == _HC - 1))
    def _finish():
        sse = jnp.sum(sse_ref[...], keepdims=True)
        loss_ref[...] = sse * ((1.0 + _CC) / (_N * _D))
        p = cnt_ref[...] * (1.0 / _N)
        ent = jnp.sum(p * jnp.log(p + 1e-10), axis=(0, 1), keepdims=True)
        perp_ref[...] = jnp.exp(-ent)


@jax.jit
def kernel(inputs, embedding):
    emb_t = embedding.T

    grid = (_B, _HC)
    loss2d, quantized, perp2d, encodings = pl.pallas_call(
        _vq_kernel,
        grid=grid,
        in_specs=[
            pl.BlockSpec((1, _D, 32 // _HC, 32), lambda b, hc: (b, 0, hc, 0)),
            pl.BlockSpec((_K, _D), lambda b, hc: (0, 0)),
            pl.BlockSpec((_D, _K), lambda b, hc: (0, 0)),
        ],
        out_specs=[
            pl.BlockSpec((1, 1), lambda b, hc: (0, 0)),
            pl.BlockSpec((1, _D, 32 // _HC, 32), lambda b, hc: (b, 0, hc, 0)),
            pl.BlockSpec((1, 1), lambda b, hc: (0, 0)),
            pl.BlockSpec((_T, _K), lambda b, hc: (b * _HC + hc, 0)),
        ],
        out_shape=[
            jax.ShapeDtypeStruct((1, 1), jnp.float32),
            jax.ShapeDtypeStruct((_B, _D, 32, 32), jnp.float32),
            jax.ShapeDtypeStruct((1, 1), jnp.float32),
            jax.ShapeDtypeStruct((_N, _K), jnp.float32),
        ],
        scratch_shapes=[
            pltpu.VMEM((1, _T), jnp.float32),
            pltpu.VMEM((_K, 1), jnp.float32),
        ],
    )(inputs, embedding, emb_t)

    return (loss2d[0, 0], quantized, perp2d[0, 0], encodings)
